# Initial kernel scaffold; baseline (speedup 1.0000x reference)
import functools

import jax
import jax.numpy as jnp
from jax import lax
from jax.experimental import pallas as pl
from jax.experimental.pallas import tpu as pltpu
from jax.experimental.pallas import tpu_sc as plsc

_N = 10000
_E = 640000
_H = 64
_G = 512
_NC = 2
_NS = 16
_NW = _NC * _NS
_K = 128
_CHUNKS = _E // _K
_T_BASE = _CHUNKS // _NW
_T_EXTRA = _CHUNKS - _T_BASE * _NW
_PAIRS = (_T_BASE + 1 + 1) // 2

_SC_MESH = plsc.VectorSubcoreMesh(core_axis_name="c", subcore_axis_name="s")
_F32 = jnp.float32


def _worker_id():
    return lax.axis_index("s") * _NC + lax.axis_index("c")


def _tile_chunk_range(wid):
    t = jnp.where(wid < _T_EXTRA, _T_BASE + 1, _T_BASE)
    base = _T_BASE * wid + jnp.minimum(wid, _T_EXTRA)
    return base, t


def _deg_body(dst2d, deg_out, idx_all, ones_v, zeros_v, deg_sh):
    c = lax.axis_index("c")
    s = lax.axis_index("s")
    wid = s * _NC + c
    base, t = _tile_chunk_range(wid)

    for i in range(8):
        ones_v[pl.ds(i * 16, 16), :] = jnp.ones((16, 1), _F32)

    def _zb(i, _):
        zeros_v[pl.ds(i * 16, 16), :] = jnp.zeros((16, 1), _F32)
        return 0
    lax.fori_loop(0, 63, _zb, 0)

    @pl.when(s < 10)
    def _():
        pltpu.sync_copy(zeros_v.at[pl.ds(0, 1000)], deg_sh.at[pl.ds(s * 1000, 1000)])

    plsc.subcore_barrier()

    pltpu.sync_copy(dst2d.at[pl.ds(base, _T_BASE + 1)], idx_all)

    def _chunk(j, _):
        pltpu.sync_copy(ones_v, deg_sh.at[idx_all.at[j]], add=True)
        return 0
    lax.fori_loop(0, t, _chunk, 0)

    plsc.subcore_barrier()

    @pl.when(s == 0)
    def _():
        pltpu.sync_copy(deg_sh, deg_out.at[c])


_deg_kernel = pl.kernel(
    _deg_body,
    out_type=jax.ShapeDtypeStruct((_NC, _N, 1), _F32),
    mesh=_SC_MESH,
    scratch_types=[
        pltpu.VMEM((_T_BASE + 1, _K), jnp.int32),
        pltpu.VMEM((_K, 1), _F32),
        pltpu.VMEM((1008, 1), _F32),
        pltpu.VMEM_SHARED((_N, 1), _F32),
    ],
)


def _msg_body(u_hbm, src2d, dst2d, s_part, sidx_all, didx_all, rows, zrow, acc_sh, gsem):
    c = lax.axis_index("c")
    s = lax.axis_index("s")
    wid = s * _NC + c
    base, t = _tile_chunk_range(wid)

    def _zb(i, _):
        for cc in range(4):
            zrow[i, pl.ds(cc * 16, 16)] = jnp.zeros((16,), _F32)
        return 0
    lax.fori_loop(0, 125, _zb, 0)

    def _zc(k, _):
        pltpu.sync_copy(zrow, acc_sh.at[pl.ds(s * 625 + k * 125, 125)])
        return 0
    lax.fori_loop(0, 5, _zc, 0)

    plsc.subcore_barrier()

    pltpu.sync_copy(src2d.at[pl.ds(base, _T_BASE + 1)], sidx_all)
    pltpu.sync_copy(dst2d.at[pl.ds(base, _T_BASE + 1)], didx_all)

    for b in range(2):
        pltpu.async_copy(u_hbm.at[sidx_all.at[b]], rows.at[b], gsem)

    def _pair(jj, _):
        for b in range(2):
            j = 2 * jj + b

            @pl.when(j < t)
            def _():
                pltpu.make_async_copy(u_hbm.at[sidx_all.at[j]], rows.at[b], gsem).wait()
                pltpu.sync_copy(rows.at[b], acc_sh.at[didx_all.at[j]], add=True)

                @pl.when(j + 2 < t)
                def _():
                    pltpu.async_copy(u_hbm.at[sidx_all.at[j + 2]], rows.at[b], gsem)
        return 0
    lax.fori_loop(0, _PAIRS, _pair, 0)

    plsc.subcore_barrier()
    pltpu.sync_copy(acc_sh.at[pl.ds(s * 625, 625)], s_part.at[c, pl.ds(s * 625, 625)])


_msg_kernel = pl.kernel(
    _msg_body,
    out_type=jax.ShapeDtypeStruct((_NC, _N, _H), _F32),
    mesh=_SC_MESH,
    scratch_types=[
        pltpu.VMEM((_T_BASE + 1, _K), jnp.int32),
        pltpu.VMEM((_T_BASE + 1, _K), jnp.int32),
        pltpu.VMEM((2, _K, _H), _F32),
        pltpu.VMEM((125, _H), _F32),
        pltpu.VMEM_SHARED((_N, _H), _F32),
        pltpu.SemaphoreType.DMA,
    ],
)


_POOL_CHUNK = 128


def _pool_body(h_pad, starts_ext, gmax_out, gsum_out, sb, hb, res_max, res_sum):
    c = lax.axis_index("c")
    s = lax.axis_index("s")
    wid = s * _NC + c

    pltpu.sync_copy(starts_ext.at[pl.ds(wid * 16, 24)], sb)

    neg_inf = jnp.full((16,), -jnp.inf, _F32)
    zero = jnp.zeros((16,), _F32)

    for gl in range(16):
        a = sb[gl]
        e = sb[gl + 1]

        def _cond(carry):
            return carry[0] < e

        def _body(carry):
            pos = carry[0]
            accs = carry[1:]
            pltpu.sync_copy(h_pad.at[pl.ds(pos, _POOL_CHUNK)], hb)
            rmax = jnp.minimum(e - pos, _POOL_CHUNK)

            def _row(r, accs):
                new = []
                for cc in range(4):
                    v = hb[r, pl.ds(cc * 16, 16)]
                    new.append(jnp.maximum(accs[cc], v))
                for cc in range(4):
                    v = hb[r, pl.ds(cc * 16, 16)]
                    new.append(accs[4 + cc] + v)
                return tuple(new)

            accs = lax.fori_loop(0, rmax, _row, tuple(accs))
            return (pos + _POOL_CHUNK,) + accs

        init = (a, neg_inf, neg_inf, neg_inf, neg_inf, zero, zero, zero, zero)
        out = lax.while_loop(_cond, _body, init)
        for cc in range(4):
            res_max[gl, pl.ds(cc * 16, 16)] = out[1 + cc]
            res_sum[gl, pl.ds(cc * 16, 16)] = out[5 + cc]

    pltpu.sync_copy(res_max, gmax_out.at[pl.ds(wid * 16, 16)])
    pltpu.sync_copy(res_sum, gsum_out.at[pl.ds(wid * 16, 16)])


_pool_kernel = pl.kernel(
    _pool_body,
    out_type=(
        jax.ShapeDtypeStruct((_G, _H), _F32),
        jax.ShapeDtypeStruct((_G, _H), _F32),
    ),
    mesh=_SC_MESH,
    scratch_types=[
        pltpu.VMEM((24,), jnp.int32),
        pltpu.VMEM((_POOL_CHUNK, _H), _F32),
        pltpu.VMEM((16, _H), _F32),
        pltpu.VMEM((16, _H), _F32),
    ],
)


def _prep_body(deg_ref, x_ref, w0_ref, batch_ref, dinv_ref, u0_ref, starts_ref, cnts_ref):
    deg = deg_ref[0] + deg_ref[1] + 1.0
    dinv = lax.rsqrt(deg)
    dinv_ref[...] = dinv
    u0_ref[...] = jnp.dot(x_ref[...], w0_ref[...], preferred_element_type=_F32) * dinv
    b = batch_ref[...]
    for chunk in range(4):
        g = lax.broadcasted_iota(jnp.int32, (128, 1), 0) + chunk * 128
        lt = (b < g).astype(jnp.int32)
        eq = (b == g).astype(jnp.int32)
        starts_ref[chunk, :] = jnp.sum(lt, axis=1)
        cnts_ref[chunk, :] = jnp.sum(eq, axis=1)


def _prep_call(deg_part, x, W0, batch2d):
    return pl.pallas_call(
        _prep_body,
        out_shape=(
            jax.ShapeDtypeStruct((_N, 1), _F32),
            jax.ShapeDtypeStruct((_N, _H), _F32),
            jax.ShapeDtypeStruct((4, 128), jnp.int32),
            jax.ShapeDtypeStruct((4, 128), jnp.int32),
        ),
    )(deg_part, x, W0, batch2d)


def _layer_body_mm(spart_ref, u_ref, dinv_ref, b_ref, w_ref, h_ref, unext_ref):
    dinv = dinv_ref[...]
    agg = (spart_ref[0] + spart_ref[1] + u_ref[...]) * dinv + b_ref[...]
    h = jnp.tanh(agg)
    h_ref[...] = h
    unext_ref[...] = jnp.dot(h, w_ref[...], preferred_element_type=_F32) * dinv


def _layer_body_last(spart_ref, u_ref, dinv_ref, b_ref, h_ref):
    agg = (spart_ref[0] + spart_ref[1] + u_ref[...]) * dinv_ref[...] + b_ref[...]
    h_ref[...] = jnp.tanh(agg)


def _layer_call(s_part, u, dinv, bvec, W_next):
    if W_next is None:
        return pl.pallas_call(
            _layer_body_last,
            out_shape=jax.ShapeDtypeStruct((_N, _H), _F32),
        )(s_part, u, dinv, bvec)
    return pl.pallas_call(
        _layer_body_mm,
        out_shape=(
            jax.ShapeDtypeStruct((_N, _H), _F32),
            jax.ShapeDtypeStruct((_N, _H), _F32),
        ),
    )(s_part, u, dinv, bvec, W_next)


def _final_body(gmax_ref, gsum_ref, cnts_ref, wout_ref, bout_ref, out_ref, pooled_ref):
    cnt = cnts_ref[...].astype(_F32)
    gmaxf = jnp.where(cnt > 0, gmax_ref[...], 0.0)
    gmean = gsum_ref[...] / jnp.maximum(cnt, 1.0)
    pooled = jnp.concatenate([gmaxf, gmean], axis=1)
    pooled_ref[...] = pooled
    out_ref[...] = jnp.dot(pooled, wout_ref[...], preferred_element_type=_F32) + bout_ref[...]


def _final_call(gmax, gsum, cnts2d, Wout, bout2d):
    return pl.pallas_call(
        _final_body,
        out_shape=(
            jax.ShapeDtypeStruct((_G, 1), _F32),
            jax.ShapeDtypeStruct((_G, 2 * _H), _F32),
        ),
    )(gmax, gsum, cnts2d, Wout, bout2d)


def kernel(x, edge_index, batch_index, W0, b0, W1, b1, W2, b2, W3, b3, Wout, bout):
    src2d = jnp.pad(edge_index[0].reshape(_CHUNKS, _K), ((0, 8), (0, 0)))
    dst2d = jnp.pad(edge_index[1].reshape(_CHUNKS, _K), ((0, 8), (0, 0)))
    batch2d = batch_index.reshape(1, _N)

    deg_part = _deg_kernel(dst2d)
    dinv, u, starts4, cnts4 = _prep_call(deg_part, x, W0, batch2d)

    biases = [b0.reshape(1, _H), b1.reshape(1, _H), b2.reshape(1, _H), b3.reshape(1, _H)]
    weights = [W1, W2, W3, None]
    h = None
    for layer in range(4):
        s_part = _msg_kernel(u, src2d, dst2d)
        res = _layer_call(s_part, u, dinv, biases[layer], weights[layer])
        if layer < 3:
            h, u = res
        else:
            h = res

    h_pad = jnp.pad(h, ((0, _POOL_CHUNK), (0, 0)))
    starts_ext = jnp.concatenate(
        [starts4.reshape(_G), jnp.full((32,), _N, jnp.int32)])
    gmax, gsum = _pool_kernel(h_pad, starts_ext)

    cnts2d = cnts4.reshape(_G, 1)
    out, pooled = _final_call(gmax, gsum, cnts2d, Wout, bout.reshape(1, 1))
    return out, pooled


# speedup vs baseline: 13.8741x; 13.8741x over previous
import jax
import jax.numpy as jnp
from jax import lax
from jax.experimental import pallas as pl
from jax.experimental.pallas import tpu as pltpu
from jax.experimental.pallas import tpu_sc as plsc

_N = 10000
_NP = 10016
_E = 640000
_H = 64
_G = 512
_NC = 2
_NS = 16
_NW = _NC * _NS
_K = 128
_T = 160
_EP = _NW * _T * _K

_SC_MESH = plsc.VectorSubcoreMesh(core_axis_name="c", subcore_axis_name="s")
_F32 = jnp.float32


def _row_part(s):
    off_small = 1264 + (s - 2) * 624
    return jnp.where(s < 2, s * 632, off_small)


def _deg_body(dst3d, deg_out, idx_all, ones_v, zeros_v, deg_tile, deg_sh):
    c = lax.axis_index("c")
    s = lax.axis_index("s")
    wid = s * _NC + c

    for i in range(8):
        ones_v[pl.ds(i * 16, 16)] = jnp.ones((16,), _F32)

    def _zb(i, _):
        zeros_v[pl.ds(i * 16, 16)] = jnp.zeros((16,), _F32)
        return 0
    lax.fori_loop(0, 63, _zb, 0)

    @pl.when(s < 10)
    def _():
        pltpu.sync_copy(zeros_v.at[pl.ds(0, 1000)], deg_sh.at[pl.ds(s * 1000, 1000)])

    plsc.subcore_barrier()

    pltpu.sync_copy(dst3d.at[wid], idx_all)

    def _chunk(j, _):
        pltpu.sync_copy(ones_v, deg_sh.at[idx_all.at[j]], add=True)
        return 0
    lax.fori_loop(0, _T, _chunk, 0)

    plsc.subcore_barrier()

    @pl.when(s == 0)
    def _():
        pltpu.sync_copy(deg_sh.at[pl.ds(0, _N)], deg_tile)
        pltpu.sync_copy(deg_tile, deg_out.at[c, 0])


_deg_kernel = pl.kernel(
    _deg_body,
    out_type=jax.ShapeDtypeStruct((_NC, 1, _N), _F32),
    mesh=_SC_MESH,
    compiler_params=pltpu.CompilerParams(use_tc_tiling_on_sc=False),
    scratch_types=[
        pltpu.VMEM((_T, _K), jnp.int32),
        pltpu.VMEM((_K,), _F32),
        pltpu.VMEM((1008,), _F32),
        pltpu.VMEM((_N,), _F32),
        pltpu.VMEM_SHARED((_NP,), _F32),
    ],
)


def _msg_body(u_hbm, src3d, dst3d, s_part, sidx_all, didx_all, rows, zrow, acc_sh, gsem):
    c = lax.axis_index("c")
    s = lax.axis_index("s")
    wid = s * _NC + c

    def _zb(i, _):
        for cc in range(4):
            zrow[i, pl.ds(cc * 16, 16)] = jnp.zeros((16,), _F32)
        return 0
    lax.fori_loop(0, 160, _zb, 0)

    off = _row_part(s)

    @pl.when(s < 2)
    def _():
        for k in range(3):
            pltpu.sync_copy(zrow, acc_sh.at[pl.ds(off + k * 160, 160)])
        pltpu.sync_copy(zrow.at[pl.ds(0, 152)], acc_sh.at[pl.ds(off + 480, 152)])
        @pl.when(s == 0)
        def _():
            pltpu.sync_copy(zrow.at[pl.ds(0, 16)], acc_sh.at[pl.ds(_N, 16)])

    @pl.when(s >= 2)
    def _():
        for k in range(3):
            pltpu.sync_copy(zrow, acc_sh.at[pl.ds(off + k * 160, 160)])
        pltpu.sync_copy(zrow.at[pl.ds(0, 144)], acc_sh.at[pl.ds(off + 480, 144)])

    plsc.subcore_barrier()

    pltpu.sync_copy(src3d.at[wid], sidx_all)
    pltpu.sync_copy(dst3d.at[wid], didx_all)

    for b in range(2):
        pltpu.async_copy(u_hbm.at[sidx_all.at[b]], rows.at[b], gsem)

    def _pair(jj, _):
        for b in range(2):
            j = 2 * jj + b
            pltpu.make_async_copy(u_hbm.at[sidx_all.at[j]], rows.at[b], gsem).wait()
            pltpu.sync_copy(rows.at[b], acc_sh.at[didx_all.at[j]], add=True)

            @pl.when(j + 2 < _T)
            def _():
                pltpu.async_copy(u_hbm.at[sidx_all.at[j + 2]], rows.at[b], gsem)
        return 0
    lax.fori_loop(0, _T // 2, _pair, 0)

    plsc.subcore_barrier()

    @pl.when(s < 2)
    def _():
        pltpu.sync_copy(acc_sh.at[pl.ds(off, 632)], s_part.at[c, pl.ds(off, 632)])

    @pl.when(s >= 2)
    def _():
        pltpu.sync_copy(acc_sh.at[pl.ds(off, 624)], s_part.at[c, pl.ds(off, 624)])


_msg_kernel = pl.kernel(
    _msg_body,
    out_type=jax.ShapeDtypeStruct((_NC, _N, _H), _F32),
    mesh=_SC_MESH,
    compiler_params=pltpu.CompilerParams(use_tc_tiling_on_sc=False),
    scratch_types=[
        pltpu.VMEM((_T, _K), jnp.int32),
        pltpu.VMEM((_T, _K), jnp.int32),
        pltpu.VMEM((2, _K, _H), _F32),
        pltpu.VMEM((160, _H), _F32),
        pltpu.VMEM_SHARED((_NP, _H), _F32),
        pltpu.SemaphoreType.DMA,
    ],
)


_POOL_CHUNK = 128


def _pool_body(h_pad, starts_win, gmax_out, gsum_out, sb, hb, res_max, res_sum):
    c = lax.axis_index("c")
    s = lax.axis_index("s")
    wid = s * _NC + c

    pltpu.sync_copy(starts_win.at[wid], sb)

    neg_inf = jnp.full((16,), -jnp.inf, _F32)
    zero = jnp.zeros((16,), _F32)

    sv_lo = sb[0, pl.ds(0, 16)]
    sv_hi = sb[0, pl.ds(8, 16)]

    for gl in range(16):
        a = sv_lo[gl]
        e = sv_lo[gl + 1] if gl < 15 else sv_hi[8]
        a0 = (a // 8) * 8
        nch = (e - a0 + _POOL_CHUNK - 1) // _POOL_CHUNK

        def _chunk(k, accs):
            pos = a0 + k * _POOL_CHUNK
            pltpu.sync_copy(h_pad.at[pl.ds(pos, _POOL_CHUNK)], hb)
            lo = jnp.maximum(a - pos, 0)
            hi = jnp.minimum(e - pos, _POOL_CHUNK)

            def _row(r, accs):
                new = []
                for cc in range(4):
                    v = hb[r, pl.ds(cc * 16, 16)]
                    new.append(jnp.maximum(accs[cc], v))
                for cc in range(4):
                    v = hb[r, pl.ds(cc * 16, 16)]
                    new.append(accs[4 + cc] + v)
                return tuple(new)

            return lax.fori_loop(lo, hi, _row, accs)

        init = (neg_inf, neg_inf, neg_inf, neg_inf, zero, zero, zero, zero)
        out = lax.fori_loop(0, nch, _chunk, init)
        for cc in range(4):
            res_max[gl, pl.ds(cc * 16, 16)] = out[cc]
            res_sum[gl, pl.ds(cc * 16, 16)] = out[4 + cc]

    pltpu.sync_copy(res_max, gmax_out.at[pl.ds(wid * 16, 16)])
    pltpu.sync_copy(res_sum, gsum_out.at[pl.ds(wid * 16, 16)])


_pool_kernel = pl.kernel(
    _pool_body,
    out_type=(
        jax.ShapeDtypeStruct((_G, _H), _F32),
        jax.ShapeDtypeStruct((_G, _H), _F32),
    ),
    mesh=_SC_MESH,
    compiler_params=pltpu.CompilerParams(use_tc_tiling_on_sc=False),
    scratch_types=[
        pltpu.VMEM((1, 24), jnp.int32),
        pltpu.VMEM((_POOL_CHUNK, _H), _F32),
        pltpu.VMEM((16, _H), _F32),
        pltpu.VMEM((16, _H), _F32),
    ],
)


def _prep_body(deg_ref, x_ref, w0_ref, batch_ref, dinv_ref, u0_ref, starts_ref, cnts_ref):
    deg_row = deg_ref[0] + deg_ref[1] + 1.0
    dinv = lax.rsqrt(jnp.transpose(deg_row))
    dinv_ref[...] = dinv
    u0_ref[...] = jnp.dot(x_ref[...], w0_ref[...], preferred_element_type=_F32) * dinv
    b = batch_ref[...]
    for chunk in range(4):
        g = lax.broadcasted_iota(jnp.int32, (128, 1), 0) + chunk * 128
        lt = (b < g).astype(jnp.int32)
        eq = (b == g).astype(jnp.int32)
        starts_ref[chunk, :] = jnp.sum(lt, axis=1)
        cnts_ref[chunk, :] = jnp.sum(eq, axis=1)


def _prep_call(deg_part, x, W0, batch2d):
    return pl.pallas_call(
        _prep_body,
        out_shape=(
            jax.ShapeDtypeStruct((_N, 1), _F32),
            jax.ShapeDtypeStruct((_N, _H), _F32),
            jax.ShapeDtypeStruct((4, 128), jnp.int32),
            jax.ShapeDtypeStruct((4, 128), jnp.int32),
        ),
    )(deg_part, x, W0, batch2d)


def _layer_body_mm(spart_ref, u_ref, dinv_ref, b_ref, w_ref, h_ref, unext_ref):
    dinv = dinv_ref[...]
    agg = (spart_ref[0] + spart_ref[1] + u_ref[...]) * dinv + b_ref[...]
    h = jnp.tanh(agg)
    h_ref[...] = h
    unext_ref[...] = jnp.dot(h, w_ref[...], preferred_element_type=_F32) * dinv


def _layer_body_last(spart_ref, u_ref, dinv_ref, b_ref, h_ref):
    agg = (spart_ref[0] + spart_ref[1] + u_ref[...]) * dinv_ref[...] + b_ref[...]
    h_ref[...] = jnp.tanh(agg)


def _layer_call(s_part, u, dinv, bvec, W_next):
    if W_next is None:
        return pl.pallas_call(
            _layer_body_last,
            out_shape=jax.ShapeDtypeStruct((_N, _H), _F32),
        )(s_part, u, dinv, bvec)
    return pl.pallas_call(
        _layer_body_mm,
        out_shape=(
            jax.ShapeDtypeStruct((_N, _H), _F32),
            jax.ShapeDtypeStruct((_N, _H), _F32),
        ),
    )(s_part, u, dinv, bvec, W_next)


def _final_body(gmax_ref, gsum_ref, cnts_ref, wout_ref, bout_ref, out_ref, pooled_ref):
    cnt = cnts_ref[...].astype(_F32)
    gmaxf = jnp.where(cnt > 0, gmax_ref[...], 0.0)
    gmean = gsum_ref[...] / jnp.maximum(cnt, 1.0)
    pooled = jnp.concatenate([gmaxf, gmean], axis=1)
    pooled_ref[...] = pooled
    out_ref[...] = jnp.dot(pooled, wout_ref[...], preferred_element_type=_F32) + bout_ref[...]


def _final_call(gmax, gsum, cnts2d, Wout, bout2d):
    return pl.pallas_call(
        _final_body,
        out_shape=(
            jax.ShapeDtypeStruct((_G, 1), _F32),
            jax.ShapeDtypeStruct((_G, 2 * _H), _F32),
        ),
    )(gmax, gsum, cnts2d, Wout, bout2d)


def kernel(x, edge_index, batch_index, W0, b0, W1, b1, W2, b2, W3, b3, Wout, bout):
    npad = _EP - _E
    src3d = jnp.concatenate(
        [edge_index[0], jnp.zeros((npad,), jnp.int32)]).reshape(_NW, _T, _K)
    dst3d = jnp.concatenate(
        [edge_index[1], jnp.full((npad,), _N, jnp.int32)]).reshape(_NW, _T, _K)
    batch2d = batch_index.reshape(1, _N)

    deg_part = _deg_kernel(dst3d)
    dinv, u, starts4, cnts4 = _prep_call(deg_part, x, W0, batch2d)

    biases = [b0.reshape(1, _H), b1.reshape(1, _H), b2.reshape(1, _H), b3.reshape(1, _H)]
    weights = [W1, W2, W3, None]
    h = None
    for layer in range(4):
        s_part = _msg_kernel(u, src3d, dst3d)
        res = _layer_call(s_part, u, dinv, biases[layer], weights[layer])
        if layer < 3:
            h, u = res
        else:
            h = res

    h_pad = jnp.pad(h, ((0, _POOL_CHUNK), (0, 0)))
    starts_full = jnp.concatenate(
        [starts4.reshape(_G), jnp.full((40,), _N, jnp.int32)])
    win_idx = (jnp.arange(_NW)[:, None] * 16 + jnp.arange(24)[None, :])
    starts_win = starts_full[win_idx].reshape(_NW, 1, 24)
    gmax, gsum = _pool_kernel(h_pad, starts_win)

    cnts2d = cnts4.reshape(_G, 1)
    out, pooled = _final_call(gmax, gsum, cnts2d, Wout, bout.reshape(1, 1))
    return out, pooled


# speedup vs baseline: 13.9466x; 1.0052x over previous
import jax
import jax.numpy as jnp
from jax import lax
from jax.experimental import pallas as pl
from jax.experimental.pallas import tpu as pltpu
from jax.experimental.pallas import tpu_sc as plsc

_N = 10000
_NP = 10016
_E = 640000
_H = 64
_G = 512
_NC = 2
_NS = 16
_NW = _NC * _NS
_K = 256
_T = 80
_EP = _NW * _T * _K

_SC_MESH = plsc.VectorSubcoreMesh(core_axis_name="c", subcore_axis_name="s")
_F32 = jnp.float32


def _row_part(s):
    off_small = 1264 + (s - 2) * 624
    return jnp.where(s < 2, s * 632, off_small)


def _deg_body(dst3d, deg_out, idx_all, ones_v, zeros_v, deg_tile, deg_sh):
    c = lax.axis_index("c")
    s = lax.axis_index("s")
    wid = s * _NC + c

    for i in range(_K // 16):
        ones_v[pl.ds(i * 16, 16)] = jnp.ones((16,), _F32)

    def _zb(i, _):
        zeros_v[pl.ds(i * 16, 16)] = jnp.zeros((16,), _F32)
        return 0
    lax.fori_loop(0, 63, _zb, 0)

    @pl.when(s < 10)
    def _():
        pltpu.sync_copy(zeros_v.at[pl.ds(0, 1000)], deg_sh.at[pl.ds(s * 1000, 1000)])

    plsc.subcore_barrier()

    pltpu.sync_copy(dst3d.at[wid], idx_all)

    def _chunk(j, _):
        pltpu.sync_copy(ones_v, deg_sh.at[idx_all.at[j]], add=True)
        return 0
    lax.fori_loop(0, _T, _chunk, 0)

    plsc.subcore_barrier()

    @pl.when(s == 0)
    def _():
        pltpu.sync_copy(deg_sh.at[pl.ds(0, _N)], deg_tile)
        pltpu.sync_copy(deg_tile, deg_out.at[c, 0])


_deg_kernel = pl.kernel(
    _deg_body,
    out_type=jax.ShapeDtypeStruct((_NC, 1, _N), _F32),
    mesh=_SC_MESH,
    compiler_params=pltpu.CompilerParams(use_tc_tiling_on_sc=False),
    scratch_types=[
        pltpu.VMEM((_T, _K), jnp.int32),
        pltpu.VMEM((_K,), _F32),
        pltpu.VMEM((1008,), _F32),
        pltpu.VMEM((_N,), _F32),
        pltpu.VMEM_SHARED((_NP,), _F32),
    ],
)


def _msg_body(u_hbm, src3d, dst3d, s_part, sidx_all, didx_all, rows, zrow, acc_sh, gsem):
    c = lax.axis_index("c")
    s = lax.axis_index("s")
    wid = s * _NC + c

    def _zb(i, _):
        for cc in range(4):
            zrow[i, pl.ds(cc * 16, 16)] = jnp.zeros((16,), _F32)
        return 0
    lax.fori_loop(0, 160, _zb, 0)

    off = _row_part(s)

    @pl.when(s < 2)
    def _():
        for k in range(3):
            pltpu.sync_copy(zrow, acc_sh.at[pl.ds(off + k * 160, 160)])
        pltpu.sync_copy(zrow.at[pl.ds(0, 152)], acc_sh.at[pl.ds(off + 480, 152)])
        @pl.when(s == 0)
        def _():
            pltpu.sync_copy(zrow.at[pl.ds(0, 16)], acc_sh.at[pl.ds(_N, 16)])

    @pl.when(s >= 2)
    def _():
        for k in range(3):
            pltpu.sync_copy(zrow, acc_sh.at[pl.ds(off + k * 160, 160)])
        pltpu.sync_copy(zrow.at[pl.ds(0, 144)], acc_sh.at[pl.ds(off + 480, 144)])

    plsc.subcore_barrier()

    pltpu.sync_copy(src3d.at[wid], sidx_all)
    pltpu.sync_copy(dst3d.at[wid], didx_all)

    for b in range(2):
        pltpu.async_copy(u_hbm.at[sidx_all.at[b]], rows.at[b], gsem)

    def _pair(qq, _):
        for b in range(2):
            j = 2 * qq + b
            pltpu.make_async_copy(u_hbm.at[sidx_all.at[j]], rows.at[b], gsem).wait()
            pltpu.sync_copy(rows.at[b], acc_sh.at[didx_all.at[j]], add=True)

            @pl.when(j + 2 < _T)
            def _():
                pltpu.async_copy(u_hbm.at[sidx_all.at[j + 2]], rows.at[b], gsem)
        return 0
    lax.fori_loop(0, _T // 2, _pair, 0)

    plsc.subcore_barrier()

    @pl.when(s < 2)
    def _():
        pltpu.sync_copy(acc_sh.at[pl.ds(off, 632)], s_part.at[c, pl.ds(off, 632)])

    @pl.when(s >= 2)
    def _():
        pltpu.sync_copy(acc_sh.at[pl.ds(off, 624)], s_part.at[c, pl.ds(off, 624)])


_msg_kernel = pl.kernel(
    _msg_body,
    out_type=jax.ShapeDtypeStruct((_NC, _N, _H), _F32),
    mesh=_SC_MESH,
    compiler_params=pltpu.CompilerParams(use_tc_tiling_on_sc=False),
    scratch_types=[
        pltpu.VMEM((_T, _K), jnp.int32),
        pltpu.VMEM((_T, _K), jnp.int32),
        pltpu.VMEM((2, _K, _H), _F32),
        pltpu.VMEM((160, _H), _F32),
        pltpu.VMEM_SHARED((_NP, _H), _F32),
        pltpu.SemaphoreType.DMA,
    ],
)


_POOL_CHUNK = 128


def _pool_body(h_pad, starts_win, gmax_out, gsum_out, sb, hb, res_max, res_sum):
    c = lax.axis_index("c")
    s = lax.axis_index("s")
    wid = s * _NC + c

    pltpu.sync_copy(starts_win.at[wid], sb)

    neg_inf = jnp.full((16,), -jnp.inf, _F32)
    zero = jnp.zeros((16,), _F32)

    sv_lo = sb[0, pl.ds(0, 16)]
    sv_hi = sb[0, pl.ds(8, 16)]

    for gl in range(16):
        a = sv_lo[gl]
        e = sv_lo[gl + 1] if gl < 15 else sv_hi[8]
        a0 = (a // 8) * 8
        nch = (e - a0 + _POOL_CHUNK - 1) // _POOL_CHUNK

        def _chunk(k, accs):
            pos = a0 + k * _POOL_CHUNK
            pltpu.sync_copy(h_pad.at[pl.ds(pos, _POOL_CHUNK)], hb)
            lo = jnp.maximum(a - pos, 0)
            hi = jnp.minimum(e - pos, _POOL_CHUNK)

            def _row(r, accs):
                new = []
                for cc in range(4):
                    v = hb[r, pl.ds(cc * 16, 16)]
                    new.append(jnp.maximum(accs[cc], v))
                for cc in range(4):
                    v = hb[r, pl.ds(cc * 16, 16)]
                    new.append(accs[4 + cc] + v)
                return tuple(new)

            return lax.fori_loop(lo, hi, _row, accs)

        init = (neg_inf, neg_inf, neg_inf, neg_inf, zero, zero, zero, zero)
        out = lax.fori_loop(0, nch, _chunk, init)
        for cc in range(4):
            res_max[gl, pl.ds(cc * 16, 16)] = out[cc]
            res_sum[gl, pl.ds(cc * 16, 16)] = out[4 + cc]

    pltpu.sync_copy(res_max, gmax_out.at[pl.ds(wid * 16, 16)])
    pltpu.sync_copy(res_sum, gsum_out.at[pl.ds(wid * 16, 16)])


_pool_kernel = pl.kernel(
    _pool_body,
    out_type=(
        jax.ShapeDtypeStruct((_G, _H), _F32),
        jax.ShapeDtypeStruct((_G, _H), _F32),
    ),
    mesh=_SC_MESH,
    compiler_params=pltpu.CompilerParams(use_tc_tiling_on_sc=False),
    scratch_types=[
        pltpu.VMEM((1, 24), jnp.int32),
        pltpu.VMEM((_POOL_CHUNK, _H), _F32),
        pltpu.VMEM((16, _H), _F32),
        pltpu.VMEM((16, _H), _F32),
    ],
)


def _prep_body(deg_ref, x_ref, w0_ref, batch_ref, dinv_ref, u0_ref, starts_ref, cnts_ref):
    deg_row = deg_ref[0] + deg_ref[1] + 1.0
    dinv = lax.rsqrt(jnp.transpose(deg_row))
    dinv_ref[...] = dinv
    u0_ref[...] = jnp.dot(x_ref[...], w0_ref[...], preferred_element_type=_F32) * dinv
    b = batch_ref[...]
    for chunk in range(4):
        g = lax.broadcasted_iota(jnp.int32, (128, 1), 0) + chunk * 128
        lt = (b < g).astype(jnp.int32)
        eq = (b == g).astype(jnp.int32)
        starts_ref[chunk, :] = jnp.sum(lt, axis=1)
        cnts_ref[chunk, :] = jnp.sum(eq, axis=1)


def _prep_call(deg_part, x, W0, batch2d):
    return pl.pallas_call(
        _prep_body,
        out_shape=(
            jax.ShapeDtypeStruct((_N, 1), _F32),
            jax.ShapeDtypeStruct((_N, _H), _F32),
            jax.ShapeDtypeStruct((4, 128), jnp.int32),
            jax.ShapeDtypeStruct((4, 128), jnp.int32),
        ),
    )(deg_part, x, W0, batch2d)


def _layer_body_mm(spart_ref, u_ref, dinv_ref, b_ref, w_ref, h_ref, unext_ref):
    dinv = dinv_ref[...]
    agg = (spart_ref[0] + spart_ref[1] + u_ref[...]) * dinv + b_ref[...]
    h = jnp.tanh(agg)
    h_ref[...] = h
    unext_ref[...] = jnp.dot(h, w_ref[...], preferred_element_type=_F32) * dinv


def _layer_body_last(spart_ref, u_ref, dinv_ref, b_ref, h_ref):
    agg = (spart_ref[0] + spart_ref[1] + u_ref[...]) * dinv_ref[...] + b_ref[...]
    h_ref[...] = jnp.tanh(agg)


def _layer_call(s_part, u, dinv, bvec, W_next):
    if W_next is None:
        return pl.pallas_call(
            _layer_body_last,
            out_shape=jax.ShapeDtypeStruct((_N, _H), _F32),
        )(s_part, u, dinv, bvec)
    return pl.pallas_call(
        _layer_body_mm,
        out_shape=(
            jax.ShapeDtypeStruct((_N, _H), _F32),
            jax.ShapeDtypeStruct((_N, _H), _F32),
        ),
    )(s_part, u, dinv, bvec, W_next)


def _final_body(gmax_ref, gsum_ref, cnts_ref, wout_ref, bout_ref, out_ref, pooled_ref):
    cnt = cnts_ref[...].astype(_F32)
    gmaxf = jnp.where(cnt > 0, gmax_ref[...], 0.0)
    gmean = gsum_ref[...] / jnp.maximum(cnt, 1.0)
    pooled = jnp.concatenate([gmaxf, gmean], axis=1)
    pooled_ref[...] = pooled
    out_ref[...] = jnp.dot(pooled, wout_ref[...], preferred_element_type=_F32) + bout_ref[...]


def _final_call(gmax, gsum, cnts2d, Wout, bout2d):
    return pl.pallas_call(
        _final_body,
        out_shape=(
            jax.ShapeDtypeStruct((_G, 1), _F32),
            jax.ShapeDtypeStruct((_G, 2 * _H), _F32),
        ),
    )(gmax, gsum, cnts2d, Wout, bout2d)


def kernel(x, edge_index, batch_index, W0, b0, W1, b1, W2, b2, W3, b3, Wout, bout):
    npad = _EP - _E
    src3d = jnp.concatenate(
        [edge_index[0], jnp.zeros((npad,), jnp.int32)]).reshape(_NW, _T, _K)
    dst3d = jnp.concatenate(
        [edge_index[1], jnp.full((npad,), _N, jnp.int32)]).reshape(_NW, _T, _K)
    batch2d = batch_index.reshape(1, _N)

    deg_part = _deg_kernel(dst3d)
    dinv, u, starts4, cnts4 = _prep_call(deg_part, x, W0, batch2d)

    biases = [b0.reshape(1, _H), b1.reshape(1, _H), b2.reshape(1, _H), b3.reshape(1, _H)]
    weights = [W1, W2, W3, None]
    h = None
    for layer in range(4):
        s_part = _msg_kernel(u, src3d, dst3d)
        res = _layer_call(s_part, u, dinv, biases[layer], weights[layer])
        if layer < 3:
            h, u = res
        else:
            h = res

    h_pad = jnp.pad(h, ((0, _POOL_CHUNK), (0, 0)))
    starts_full = jnp.concatenate(
        [starts4.reshape(_G), jnp.full((40,), _N, jnp.int32)])
    win_idx = (jnp.arange(_NW)[:, None] * 16 + jnp.arange(24)[None, :])
    starts_win = starts_full[win_idx].reshape(_NW, 1, 24)
    gmax, gsum = _pool_kernel(h_pad, starts_win)

    cnts2d = cnts4.reshape(_G, 1)
    out, pooled = _final_call(gmax, gsum, cnts2d, Wout, bout.reshape(1, 1))
    return out, pooled
